# baseline (speedup 1.0000x reference)
import functools

import jax
import jax.numpy as jnp
from jax import lax
from jax.experimental import pallas as pl
from jax.experimental.pallas import tpu as pltpu
from jax.experimental.pallas import tpu_sc as plsc


def _gather_sum(xf, table_i32, B, CH, CL, D):
  info = plsc.get_sparse_core_info()
  NC, NS = info.num_cores, info.num_subcores
  NW = NC * NS
  assert B % NW == 0
  SPW = B // NW
  NV = D // 16
  NSLOT = 4
  NU = SPW * CH
  assert NU % NSLOT == 0

  mesh = plsc.VectorSubcoreMesh(core_axis_name="c", subcore_axis_name="s")

  @functools.partial(
      pl.kernel,
      mesh=mesh,
      compiler_params=pltpu.CompilerParams(
          needs_layout_passes=False, use_tc_tiling_on_sc=False),
      out_type=jax.ShapeDtypeStruct((B, D), jnp.float32),
      scratch_types=[
          pltpu.VMEM((SPW * CH * CL,), jnp.int32),
          *[pltpu.VMEM((CL, D // 2), jnp.int32) for _ in range(NSLOT)],
          pltpu.VMEM((SPW, D), jnp.float32),
          pltpu.SemaphoreType.DMA((NSLOT,)),
      ],
  )
  def k(x_hbm, table_hbm, out_hbm, idx_v, r0, r1, r2, r3, acc_v, sems):
    rows = (r0, r1, r2, r3)
    cid = lax.axis_index("c")
    sid = lax.axis_index("s")
    wid = sid * NC + cid
    base = wid * SPW

    pltpu.sync_copy(x_hbm.at[pl.ds(base * CH * CL, SPW * CH * CL)], idx_v)
    zero = jnp.zeros((16,), jnp.float32)

    def fire(u, sl):
      pltpu.async_copy(
          table_hbm.at[idx_v.at[pl.ds(u * CL, CL)]],
          rows[sl],
          sems.at[sl],
      )

    for sl in range(NSLOT):
      fire(sl, sl)

    def group(g, acc):
      for sl in range(NSLOT):
        u = g * NSLOT + sl
        pltpu.make_async_copy(
            table_hbm.at[idx_v.at[pl.ds(0, CL)]],
            rows[sl],
            sems.at[sl],
        ).wait()

        out = list(acc)
        for r in range(CL):
          for j in range(NV // 2):
            w = rows[sl][r, pl.ds(j * 16, 16)]
            ev = plsc.bitcast(w << 16, jnp.float32)
            od = plsc.bitcast(w & jnp.int32(-65536), jnp.float32)
            out[2 * j] = out[2 * j] + ev
            out[2 * j + 1] = out[2 * j + 1] + od

        is_last = u % CH == CH - 1
        si = u // CH

        @pl.when(is_last)
        def _():
          for j in range(NV):
            acc_v[si, pl.ds(j * 16, 16)] = out[j]

        @pl.when(u + NSLOT < NU)
        def _():
          fire(u + NSLOT, sl)

        keep = jnp.where(is_last, jnp.float32(0.0), jnp.float32(1.0))
        acc = tuple(out[j] * keep for j in range(NV))
      return acc

    lax.fori_loop(0, NU // NSLOT, group, (zero,) * NV)
    pltpu.sync_copy(acc_v, out_hbm.at[pl.ds(base, SPW)])

  return k(xf, table_i32)


def _mlp_body(rep_ref, len_ref, w1t_ref, b1_ref, w2t_ref, b2_ref, out_ref):
  rep = rep_ref[...] / len_ref[...]
  h = jnp.dot(rep, w1t_ref[...], preferred_element_type=jnp.float32)
  h = jnp.maximum(h + b1_ref[...], 0.0)
  out = jnp.dot(h, w2t_ref[...], preferred_element_type=jnp.float32)
  out_ref[...] = out + b2_ref[...]


def kernel(x, lengths, table, W1, b1, W2, b2):
  B, L = x.shape
  D = table.shape[1]
  H = W1.shape[0]
  O = W2.shape[0]

  CL = max((r for r in range(8, 65, 8) if L % r == 0), default=L)
  CH = L // CL
  xf = x.reshape(B * L)

  tb32 = jax.lax.bitcast_convert_type(
      table.astype(jnp.bfloat16).reshape(table.shape[0], D // 2, 2),
      jnp.int32)
  rep_sum = _gather_sum(xf, tb32, B, CH, CL, D)

  blk = jnp.arange(0, D, 32)[:, None, None]
  within = jnp.arange(16)[None, None, :]
  d_perm = (blk + 2 * within + jnp.array([0, 1])[None, :, None]).reshape(-1)

  lens = lengths.astype(jnp.float32).reshape(B, 1)
  logits = pl.pallas_call(
      _mlp_body,
      out_shape=jax.ShapeDtypeStruct((B, O), jnp.float32),
  )(rep_sum, lens, W1.T[d_perm], b1.reshape(1, H), W2.T, b2.reshape(1, O))
  return logits


# speedup vs baseline: 3.5643x; 3.5643x over previous
import functools

import jax
import jax.numpy as jnp
from jax import lax
from jax.experimental import pallas as pl
from jax.experimental.pallas import tpu as pltpu
from jax.experimental.pallas import tpu_sc as plsc


def _gather_sum(x3, table, B, CH, CL, D):
  info = plsc.get_sparse_core_info()
  NC, NS = info.num_cores, info.num_subcores
  NW = NC * NS
  assert B % NW == 0
  SPW = B // NW
  assert SPW % 2 == 0
  NV = D // 16
  NSLOT = 8

  mesh = plsc.VectorSubcoreMesh(core_axis_name="c", subcore_axis_name="s")

  @functools.partial(
      pl.kernel,
      mesh=mesh,
      out_type=jax.ShapeDtypeStruct((B, D), jnp.float32),
      scratch_types=[
          pltpu.VMEM((SPW * CH * CL,), jnp.int32),
          pltpu.VMEM((NSLOT, CL, D), jnp.float32),
          pltpu.VMEM((SPW, D), jnp.float32),
          pltpu.SemaphoreType.DMA((NSLOT,)),
      ],
  )
  def k(x_hbm, table_hbm, out_hbm, idx_v, rows_v, acc_v, sems):
    cid = lax.axis_index("c")
    sid = lax.axis_index("s")
    wid = sid * NC + cid
    base = wid * SPW

    pltpu.sync_copy(x_hbm.at[pl.ds(base * CH * CL, SPW * CH * CL)], idx_v)
    zero = jnp.zeros((16,), jnp.float32)

    def zero_buf(slot):
      def zb(r, c):
        for j in range(NV):
          rows_v[slot, r, pl.ds(j * 16, 16)] = zero
        return c

      lax.fori_loop(0, CL, zb, 0)

    for sl in range(NSLOT):
      zero_buf(sl)

    def step(i, carry):
      slot = i % NSLOT

      @pl.when(i >= NSLOT)
      def _():
        si = i - NSLOT

        def db(ch, c):
          pltpu.make_async_copy(
              table_hbm.at[idx_v.at[pl.ds(0, CL)]],
              rows_v.at[slot],
              sems.at[slot],
          ).wait()
          return c

        lax.fori_loop(0, CH, db, 0)

        def body(r, acc):
          out = []
          for j in range(NV):
            out.append(acc[j] + rows_v[slot, r, pl.ds(j * 16, 16)])
          for j in range(NV):
            rows_v[slot, r, pl.ds(j * 16, 16)] = zero
          return tuple(out)

        acc = lax.fori_loop(0, CL, body, (zero,) * NV)
        for j in range(NV):
          acc_v[si, pl.ds(j * 16, 16)] = acc[j]

      @pl.when(i < SPW)
      def _():
        def fb(ch, c):
          pltpu.async_copy(
              table_hbm.at[idx_v.at[pl.ds((i * CH + ch) * CL, CL)]],
              rows_v.at[slot],
              sems.at[slot],
              add=True,
          )
          return c

        lax.fori_loop(0, CH, fb, 0)

      return carry

    lax.fori_loop(0, SPW + NSLOT, step, 0)
    pltpu.sync_copy(acc_v, out_hbm.at[pl.ds(base, SPW)])

  return k(x3, table)


def _mlp_body(rep_ref, len_ref, w1t_ref, b1_ref, w2t_ref, b2_ref, out_ref):
  rep = rep_ref[...] / len_ref[...]
  h = jnp.dot(rep, w1t_ref[...], preferred_element_type=jnp.float32)
  h = jnp.maximum(h + b1_ref[...], 0.0)
  out = jnp.dot(h, w2t_ref[...], preferred_element_type=jnp.float32)
  out_ref[...] = out + b2_ref[...]


def kernel(x, lengths, table, W1, b1, W2, b2):
  B, L = x.shape
  D = table.shape[1]
  H = W1.shape[0]
  O = W2.shape[0]

  CL = max((r for r in range(8, 65, 8) if L % r == 0), default=L)
  CH = L // CL
  x3 = x.reshape(B * L)

  rep_sum = _gather_sum(x3, table, B, CH, CL, D)

  lens = lengths.astype(jnp.float32).reshape(B, 1)
  logits = pl.pallas_call(
      _mlp_body,
      out_shape=jax.ShapeDtypeStruct((B, O), jnp.float32),
  )(rep_sum, lens, W1.T, b1.reshape(1, H), W2.T, b2.reshape(1, O))
  return logits
